# baseline (speedup 1.0000x reference)
import functools

import jax
import jax.numpy as jnp
from jax import lax
from jax.experimental import pallas as pl
from jax.experimental.pallas import tpu as pltpu
from jax.experimental.pallas import tpu_sc as plsc

CORPUS_SIZE = 100000
EMBEDDING_DIM = 100
BATCH = 1024
SEQ_LEN = 20

_INFO = plsc.get_sparse_core_info()
NC = _INFO.num_cores
NS = _INFO.num_subcores
LANES = _INFO.num_lanes
NW = NC * NS

BLOCK_ROWS = 128
NUM_BLOCKS = -(-CORPUS_SIZE // BLOCK_ROWS)
SCALARS_LEN = -(-NUM_BLOCKS * BLOCK_ROWS // 128) * 128
GROUPS = BLOCK_ROWS // LANES
MAX_NB = -(-NUM_BLOCKS // NW)
FULL_NB = MAX_NB - 1
N_EXTRA = NUM_BLOCKS - FULL_NB * NW

SENT_PER_W = BATCH // NW
TOK_PER_W = SENT_PER_W * SEQ_LEN
TOK_GROUPS = TOK_PER_W // LANES
IDX_CHUNK = 128
N_CHUNKS = TOK_PER_W // IDX_CHUNK


def _newton_sqrt(x):
    i = lax.bitcast_convert_type(x, jnp.int32)
    y = lax.bitcast_convert_type(jnp.int32(0x5F3759DF) - (i >> 1), jnp.float32)
    for _ in range(3):
        y = y * (1.5 - 0.5 * x * y * y)
    return x * y


def _make_scan_kernel():
    mesh = plsc.VectorSubcoreMesh(core_axis_name="c", subcore_axis_name="s")

    @functools.partial(
        pl.kernel,
        mesh=mesh,
        out_type=(
            jax.ShapeDtypeStruct((SCALARS_LEN,), jnp.float32),
            jax.ShapeDtypeStruct((SCALARS_LEN,), jnp.float32),
            jax.ShapeDtypeStruct((BATCH * SEQ_LEN,), jnp.int32),
        ),
        compiler_params=pltpu.CompilerParams(
            needs_layout_passes=False, use_tc_tiling_on_sc=True
        ),
        scratch_types=[
            pltpu.VMEM((BLOCK_ROWS, EMBEDDING_DIM), jnp.float32),
            pltpu.VMEM((BLOCK_ROWS, EMBEDDING_DIM), jnp.float32),
            pltpu.VMEM((128,), jnp.float32),
            pltpu.VMEM((128,), jnp.float32),
            pltpu.VMEM((BLOCK_ROWS,), jnp.float32),
            pltpu.VMEM((BLOCK_ROWS,), jnp.float32),
            pltpu.VMEM((BLOCK_ROWS,), jnp.float32),
            pltpu.VMEM((BLOCK_ROWS,), jnp.float32),
            pltpu.VMEM((SENT_PER_W, SEQ_LEN), jnp.int32),
            pltpu.VMEM((TOK_PER_W,), jnp.int32),
            pltpu.SemaphoreType.DMA,
            pltpu.SemaphoreType.DMA,
            pltpu.SemaphoreType.DMA,
            pltpu.SemaphoreType.DMA,
        ],
    )
    def ka(table_hbm, u_hbm, w_hbm, idx2d_hbm, sq_hbm, dot_hbm, idxf_hbm,
           rows0, rows1, u_v, w_v, sq_st0, sq_st1, dot_st0, dot_st1,
           idx2_v, idx_st, sem_in0, sem_in1, sem_out0, sem_out1):
        wid = lax.axis_index("s") * NC + lax.axis_index("c")
        pltpu.sync_copy(u_hbm, u_v.at[pl.ds(0, EMBEDDING_DIM)])
        pltpu.sync_copy(w_hbm, w_v.at[pl.ds(0, EMBEDDING_DIM)])
        wrap_idx = jnp.full((LANES,), EMBEDDING_DIM, jnp.int32) + lax.iota(
            jnp.int32, LANES
        )
        plsc.store_scatter(u_v, [wrap_idx], u_v[pl.ds(0, LANES)])
        plsc.store_scatter(w_v, [wrap_idx], w_v[pl.ds(0, LANES)])

        lane_iota0 = lax.iota(jnp.int32, LANES)
        pltpu.sync_copy(
            idx2d_hbm.at[pl.ds(wid * SENT_PER_W, SENT_PER_W)], idx2_v
        )
        for g in range(TOK_PER_W // LANES):
            t = jnp.full((LANES,), g * LANES, jnp.int32) + lane_iota0
            s = t // SEQ_LEN
            j = t - s * SEQ_LEN
            idx_st[pl.ds(g * LANES, LANES)] = plsc.load_gather(idx2_v, [s, j])
        pltpu.sync_copy(idx_st, idxf_hbm.at[pl.ds(wid * TOK_PER_W, TOK_PER_W)])

        rows = (rows0, rows1)
        sq_st = (sq_st0, sq_st1)
        dot_st = (dot_st0, dot_st1)
        sem_in = (sem_in0, sem_in1)
        sem_out = (sem_out0, sem_out1)

        lane_iota = lax.iota(jnp.int32, LANES)
        row_bases = [
            jnp.full((LANES,), g * LANES, jnp.int32) + lane_iota
            for g in range(GROUPS)
        ]
        nb = jnp.where(wid < N_EXTRA, MAX_NB, FULL_NB)

        def in_base(b):
            return jnp.minimum((wid + b * NW) * BLOCK_ROWS,
                               CORPUS_SIZE - BLOCK_ROWS)

        def start_in(b, k):
            pltpu.async_copy(
                table_hbm.at[pl.ds(in_base(b), BLOCK_ROWS)], rows[k], sem_in[k]
            )

        start_in(0, 0)
        start_in(1, 1)

        def compute_block(b, k):
            base = in_base(b)
            pltpu.make_async_copy(
                table_hbm.at[pl.ds(base, BLOCK_ROWS)], rows[k], sem_in[k]
            ).wait()

            zero = jnp.zeros((LANES,), jnp.float32)
            init = (tuple(zero for _ in range(GROUPS)),
                    tuple(zero for _ in range(GROUPS)),
                    jnp.zeros((LANES,), jnp.int32))

            @plsc.parallel_loop(0, EMBEDDING_DIM, carry=init, unroll=2)
            def dim_loop(d, carry):
                sqs, dots, dvec = carry
                colv = dvec + lane_iota
                colw = jnp.where(
                    colv >= EMBEDDING_DIM, colv - EMBEDDING_DIM, colv
                )
                u_d = plsc.load_gather(u_v, [colv])
                w_d = plsc.load_gather(w_v, [colv])
                new_sqs, new_dots = [], []
                for g in range(GROUPS):
                    x = plsc.load_gather(rows[k], [row_bases[g], colw])
                    diff = u_d - x
                    new_sqs.append(sqs[g] + diff * diff)
                    new_dots.append(dots[g] + w_d * x)
                return tuple(new_sqs), tuple(new_dots), dvec + 1

            sqs, dots, _ = dim_loop

            @pl.when(b >= 2)
            def _():
                pltpu.make_async_copy(
                    sq_st[k], sq_hbm.at[pl.ds(base, BLOCK_ROWS)], sem_out[k]
                ).wait()
                pltpu.make_async_copy(
                    dot_st[k], dot_hbm.at[pl.ds(base, BLOCK_ROWS)], sem_out[k]
                ).wait()

            for g in range(GROUPS):
                sq_st[k][pl.ds(g * LANES, LANES)] = sqs[g]
                dot_st[k][pl.ds(g * LANES, LANES)] = dots[g]
            pltpu.async_copy(
                sq_st[k], sq_hbm.at[pl.ds(base, BLOCK_ROWS)], sem_out[k]
            )
            pltpu.async_copy(
                dot_st[k], dot_hbm.at[pl.ds(base, BLOCK_ROWS)], sem_out[k]
            )
            @pl.when(b + 2 < nb)
            def _():
                start_in(b + 2, k)

        def pair_body(b2, _):
            b = b2 * 2

            @pl.when(b < nb)
            def _():
                compute_block(b, 0)

            @pl.when(b + 1 < nb)
            def _():
                compute_block(b + 1, 1)
            return 0

        lax.fori_loop(0, (MAX_NB + 1) // 2, pair_body, 0)

        for k in range(2):
            pltpu.make_async_copy(
                sq_st[k], sq_hbm.at[pl.ds(in_base(0), BLOCK_ROWS)], sem_out[k]
            ).wait()
            pltpu.make_async_copy(
                dot_st[k], dot_hbm.at[pl.ds(in_base(0), BLOCK_ROWS)], sem_out[k]
            ).wait()

    return ka


def _make_pool_kernel():
    mesh = plsc.VectorSubcoreMesh(core_axis_name="c", subcore_axis_name="s")

    @functools.partial(
        pl.kernel,
        mesh=mesh,
        out_type=jax.ShapeDtypeStruct((BATCH,), jnp.float32),
        compiler_params=pltpu.CompilerParams(
            needs_layout_passes=False, use_tc_tiling_on_sc=False
        ),
        scratch_types=[
            pltpu.VMEM((N_CHUNKS, IDX_CHUNK), jnp.int32),
            pltpu.VMEM((TOK_PER_W,), jnp.float32),
            pltpu.VMEM((TOK_PER_W,), jnp.float32),
            pltpu.VMEM((TOK_PER_W,), jnp.float32),
            pltpu.VMEM((TOK_PER_W,), jnp.float32),
            pltpu.VMEM((SENT_PER_W,), jnp.float32),
            pltpu.SemaphoreType.DMA,
        ],
    )
    def kb(idx_hbm, sq_hbm, dot_hbm, out_hbm,
           idx_v, sqs_v, dots_v, alphas_v, num_v, res_v, sem):
        wid = lax.axis_index("s") * NC + lax.axis_index("c")
        for c in range(N_CHUNKS):
            pltpu.sync_copy(
                idx_hbm.at[pl.ds(wid * TOK_PER_W + c * IDX_CHUNK, IDX_CHUNK)],
                idx_v.at[c],
            )
        copies = []
        for c in range(N_CHUNKS):
            copies.append(pltpu.async_copy(
                sq_hbm.at[idx_v.at[c]],
                sqs_v.at[pl.ds(c * IDX_CHUNK, IDX_CHUNK)], sem))
            copies.append(pltpu.async_copy(
                dot_hbm.at[idx_v.at[c]],
                dots_v.at[pl.ds(c * IDX_CHUNK, IDX_CHUNK)], sem))
        for cp in copies:
            cp.wait()

        for g in range(TOK_GROUPS):
            sl = pl.ds(g * LANES, LANES)
            sq = jnp.maximum(sqs_v[sl], 1e-12)
            a = jnp.exp(_newton_sqrt(sq))
            alphas_v[sl] = a
            num_v[sl] = a * dots_v[sl]

        lane_iota = lax.iota(jnp.int32, LANES)
        for half in range(SENT_PER_W // LANES):
            sent = jnp.full((LANES,), half * LANES, jnp.int32) + lane_iota
            acc_a = jnp.zeros((LANES,), jnp.float32)
            acc_n = jnp.zeros((LANES,), jnp.float32)
            for j in range(SEQ_LEN):
                tok = sent * SEQ_LEN + j
                acc_a = acc_a + plsc.load_gather(alphas_v, [tok])
                acc_n = acc_n + plsc.load_gather(num_v, [tok])
            res_v[pl.ds(half * LANES, LANES)] = acc_n / acc_a

        pltpu.sync_copy(res_v, out_hbm.at[pl.ds(wid * SENT_PER_W, SENT_PER_W)])

    return kb


_scan_kernel = _make_scan_kernel()
_pool_kernel = _make_pool_kernel()


def kernel(batch_word_idxs, word_embeddings, weights, attend_u):
    w_flat = weights.reshape(EMBEDDING_DIM)
    sq_all, dot_all, idx_flat = _scan_kernel(
        word_embeddings, attend_u, w_flat, batch_word_idxs
    )
    out = _pool_kernel(idx_flat, sq_all, dot_all)
    return out.reshape(BATCH, 1)


# speedup vs baseline: 1.7419x; 1.7419x over previous
import functools

import jax
import jax.numpy as jnp
from jax import lax
from jax.experimental import pallas as pl
from jax.experimental.pallas import tpu as pltpu
from jax.experimental.pallas import tpu_sc as plsc

CORPUS_SIZE = 100000
EMBEDDING_DIM = 100
BATCH = 1024
SEQ_LEN = 20

_INFO = plsc.get_sparse_core_info()
NC = _INFO.num_cores
NS = _INFO.num_subcores
LANES = _INFO.num_lanes
NW = NC * NS

BLOCK_ROWS = 128
NUM_BLOCKS = -(-CORPUS_SIZE // BLOCK_ROWS)
SCALARS_LEN = -(-NUM_BLOCKS * BLOCK_ROWS // 128) * 128
GROUPS = BLOCK_ROWS // LANES
TAIL_BASE = (NUM_BLOCKS - 1) * BLOCK_ROWS
TAIL_COLS = CORPUS_SIZE - TAIL_BASE
MAX_NB = -(-NUM_BLOCKS // NW)
FULL_NB = MAX_NB - 1
N_EXTRA = NUM_BLOCKS - FULL_NB * NW

SENT_PER_W = BATCH // NW
TOK_PER_W = SENT_PER_W * SEQ_LEN
TOK_GROUPS = TOK_PER_W // LANES
IDX_CHUNK = 128
N_CHUNKS = TOK_PER_W // IDX_CHUNK


def _newton_sqrt(x):
    i = lax.bitcast_convert_type(x, jnp.int32)
    y = lax.bitcast_convert_type(jnp.int32(0x5F3759DF) - (i >> 1), jnp.float32)
    for _ in range(3):
        y = y * (1.5 - 0.5 * x * y * y)
    return x * y


def _make_scan_kernel():
    mesh = plsc.VectorSubcoreMesh(core_axis_name="c", subcore_axis_name="s")

    @functools.partial(
        pl.kernel,
        mesh=mesh,
        out_type=(
            jax.ShapeDtypeStruct((SCALARS_LEN,), jnp.float32),
            jax.ShapeDtypeStruct((SCALARS_LEN,), jnp.float32),
            jax.ShapeDtypeStruct((BATCH * SEQ_LEN,), jnp.int32),
        ),
        compiler_params=pltpu.CompilerParams(
            needs_layout_passes=False, use_tc_tiling_on_sc=True
        ),
        scratch_types=[
            pltpu.VMEM((EMBEDDING_DIM, BLOCK_ROWS), jnp.float32),
            pltpu.VMEM((EMBEDDING_DIM, BLOCK_ROWS), jnp.float32),
            pltpu.VMEM((EMBEDDING_DIM, TAIL_COLS), jnp.float32),
            pltpu.VMEM((EMBEDDING_DIM, LANES), jnp.float32),
            pltpu.VMEM((EMBEDDING_DIM, LANES), jnp.float32),
            pltpu.VMEM((EMBEDDING_DIM,), jnp.float32),
            pltpu.VMEM((EMBEDDING_DIM,), jnp.float32),
            pltpu.VMEM((BLOCK_ROWS,), jnp.float32),
            pltpu.VMEM((BLOCK_ROWS,), jnp.float32),
            pltpu.VMEM((BLOCK_ROWS,), jnp.float32),
            pltpu.VMEM((BLOCK_ROWS,), jnp.float32),
            pltpu.VMEM((SENT_PER_W, SEQ_LEN), jnp.int32),
            pltpu.VMEM((TOK_PER_W,), jnp.int32),
            pltpu.SemaphoreType.DMA,
            pltpu.SemaphoreType.DMA,
            pltpu.SemaphoreType.DMA,
            pltpu.SemaphoreType.DMA,
        ],
    )
    def ka(table_hbm, tail_hbm, u_hbm, w_hbm, idx2d_hbm,
           sq_hbm, dot_hbm, idxf_hbm,
           rows0, rows1, tail_v, usplat_v, wsplat_v, u_v, w_v,
           sq_st0, sq_st1, dot_st0, dot_st1,
           idx2_v, idx_st, sem_in0, sem_in1, sem_out0, sem_out1):
        wid = lax.axis_index("s") * NC + lax.axis_index("c")
        pltpu.sync_copy(u_hbm, u_v)
        pltpu.sync_copy(w_hbm, w_v)

        @plsc.parallel_loop(0, EMBEDDING_DIM)
        def _(d):
            dsplat = jnp.full((LANES,), d, jnp.int32)
            usplat_v[d, pl.ds(0, LANES)] = plsc.load_gather(u_v, [dsplat])
            wsplat_v[d, pl.ds(0, LANES)] = plsc.load_gather(w_v, [dsplat])

        lane_iota0 = lax.iota(jnp.int32, LANES)
        pltpu.sync_copy(
            idx2d_hbm.at[pl.ds(wid * SENT_PER_W, SENT_PER_W)], idx2_v
        )
        for g in range(TOK_PER_W // LANES):
            t = jnp.full((LANES,), g * LANES, jnp.int32) + lane_iota0
            s = t // SEQ_LEN
            j = t - s * SEQ_LEN
            idx_st[pl.ds(g * LANES, LANES)] = plsc.load_gather(idx2_v, [s, j])
        pltpu.sync_copy(idx_st, idxf_hbm.at[pl.ds(wid * TOK_PER_W, TOK_PER_W)])

        rows = (rows0, rows1)
        sq_st = (sq_st0, sq_st1)
        dot_st = (dot_st0, dot_st1)
        sem_in = (sem_in0, sem_in1)
        sem_out = (sem_out0, sem_out1)

        nb = jnp.where(wid < N_EXTRA, MAX_NB, FULL_NB)

        def start_in(b, k):
            block = wid + b * NW
            base = block * BLOCK_ROWS

            @pl.when(block < NUM_BLOCKS - 1)
            def _():
                pltpu.async_copy(
                    table_hbm.at[:, pl.ds(base, BLOCK_ROWS)], rows[k],
                    sem_in[k],
                )

            @pl.when(block == NUM_BLOCKS - 1)
            def _():
                pltpu.async_copy(tail_hbm, tail_v, sem_in[k])

        start_in(0, 0)
        start_in(1, 1)

        def sweep(buf, ngroups):
            zero = jnp.zeros((LANES,), jnp.float32)
            init = (tuple(zero for _ in range(ngroups)),
                    tuple(zero for _ in range(ngroups)))

            @plsc.parallel_loop(0, EMBEDDING_DIM, carry=init, unroll=2)
            def dim_loop(d, carry):
                sqs, dots = carry
                u_d = usplat_v[d, pl.ds(0, LANES)]
                w_d = wsplat_v[d, pl.ds(0, LANES)]
                new_sqs, new_dots = [], []
                for g in range(ngroups):
                    x = buf[d, pl.ds(g * LANES, LANES)]
                    diff = u_d - x
                    new_sqs.append(sqs[g] + diff * diff)
                    new_dots.append(dots[g] + w_d * x)
                return tuple(new_sqs), tuple(new_dots)

            return dim_loop

        def compute_block(b, k):
            block = wid + b * NW
            base = block * BLOCK_ROWS

            @pl.when(b >= 2)
            def _():
                pltpu.make_async_copy(
                    sq_st[k], sq_hbm.at[pl.ds(base, BLOCK_ROWS)], sem_out[k]
                ).wait()
                pltpu.make_async_copy(
                    dot_st[k], dot_hbm.at[pl.ds(base, BLOCK_ROWS)], sem_out[k]
                ).wait()

            @pl.when(block < NUM_BLOCKS - 1)
            def _():
                pltpu.make_async_copy(
                    table_hbm.at[:, pl.ds(base, BLOCK_ROWS)], rows[k],
                    sem_in[k],
                ).wait()
                sqs, dots = sweep(rows[k], GROUPS)
                for g in range(GROUPS):
                    sq_st[k][pl.ds(g * LANES, LANES)] = sqs[g]
                    dot_st[k][pl.ds(g * LANES, LANES)] = dots[g]

            @pl.when(block == NUM_BLOCKS - 1)
            def _():
                pltpu.make_async_copy(tail_hbm, tail_v, sem_in[k]).wait()
                sqs, dots = sweep(tail_v, TAIL_COLS // LANES)
                for g in range(TAIL_COLS // LANES):
                    sq_st[k][pl.ds(g * LANES, LANES)] = sqs[g]
                    dot_st[k][pl.ds(g * LANES, LANES)] = dots[g]

            pltpu.async_copy(
                sq_st[k], sq_hbm.at[pl.ds(base, BLOCK_ROWS)], sem_out[k]
            )
            pltpu.async_copy(
                dot_st[k], dot_hbm.at[pl.ds(base, BLOCK_ROWS)], sem_out[k]
            )
            @pl.when(b + 2 < nb)
            def _():
                start_in(b + 2, k)

        def pair_body(b2, _):
            b = b2 * 2

            @pl.when(b < nb)
            def _():
                compute_block(b, 0)

            @pl.when(b + 1 < nb)
            def _():
                compute_block(b + 1, 1)
            return 0

        lax.fori_loop(0, (MAX_NB + 1) // 2, pair_body, 0)

        for k in range(2):
            pltpu.make_async_copy(
                sq_st[k], sq_hbm.at[pl.ds(wid * BLOCK_ROWS, BLOCK_ROWS)],
                sem_out[k],
            ).wait()
            pltpu.make_async_copy(
                dot_st[k], dot_hbm.at[pl.ds(wid * BLOCK_ROWS, BLOCK_ROWS)],
                sem_out[k],
            ).wait()

    return ka


def _make_pool_kernel():
    mesh = plsc.VectorSubcoreMesh(core_axis_name="c", subcore_axis_name="s")

    @functools.partial(
        pl.kernel,
        mesh=mesh,
        out_type=jax.ShapeDtypeStruct((BATCH,), jnp.float32),
        compiler_params=pltpu.CompilerParams(
            needs_layout_passes=False, use_tc_tiling_on_sc=False
        ),
        scratch_types=[
            pltpu.VMEM((N_CHUNKS, IDX_CHUNK), jnp.int32),
            pltpu.VMEM((TOK_PER_W,), jnp.float32),
            pltpu.VMEM((TOK_PER_W,), jnp.float32),
            pltpu.VMEM((TOK_PER_W,), jnp.float32),
            pltpu.VMEM((TOK_PER_W,), jnp.float32),
            pltpu.VMEM((SENT_PER_W,), jnp.float32),
            pltpu.SemaphoreType.DMA,
        ],
    )
    def kb(idx_hbm, sq_hbm, dot_hbm, out_hbm,
           idx_v, sqs_v, dots_v, alphas_v, num_v, res_v, sem):
        wid = lax.axis_index("s") * NC + lax.axis_index("c")
        for c in range(N_CHUNKS):
            pltpu.sync_copy(
                idx_hbm.at[pl.ds(wid * TOK_PER_W + c * IDX_CHUNK, IDX_CHUNK)],
                idx_v.at[c],
            )
        copies = []
        for c in range(N_CHUNKS):
            copies.append(pltpu.async_copy(
                sq_hbm.at[idx_v.at[c]],
                sqs_v.at[pl.ds(c * IDX_CHUNK, IDX_CHUNK)], sem))
            copies.append(pltpu.async_copy(
                dot_hbm.at[idx_v.at[c]],
                dots_v.at[pl.ds(c * IDX_CHUNK, IDX_CHUNK)], sem))
        for cp in copies:
            cp.wait()

        for g in range(TOK_GROUPS):
            sl = pl.ds(g * LANES, LANES)
            sq = jnp.maximum(sqs_v[sl], 1e-12)
            a = jnp.exp(_newton_sqrt(sq))
            alphas_v[sl] = a
            num_v[sl] = a * dots_v[sl]

        lane_iota = lax.iota(jnp.int32, LANES)
        for half in range(SENT_PER_W // LANES):
            sent = jnp.full((LANES,), half * LANES, jnp.int32) + lane_iota
            acc_a = jnp.zeros((LANES,), jnp.float32)
            acc_n = jnp.zeros((LANES,), jnp.float32)
            for j in range(SEQ_LEN):
                tok = sent * SEQ_LEN + j
                acc_a = acc_a + plsc.load_gather(alphas_v, [tok])
                acc_n = acc_n + plsc.load_gather(num_v, [tok])
            res_v[pl.ds(half * LANES, LANES)] = acc_n / acc_a

        pltpu.sync_copy(res_v, out_hbm.at[pl.ds(wid * SENT_PER_W, SENT_PER_W)])

    return kb


_scan_kernel = _make_scan_kernel()
_pool_kernel = _make_pool_kernel()


def kernel(batch_word_idxs, word_embeddings, weights, attend_u):
    w_flat = weights.reshape(EMBEDDING_DIM)
    table_t = word_embeddings.T
    sq_all, dot_all, idx_flat = _scan_kernel(
        table_t, table_t[:, TAIL_BASE:], attend_u, w_flat, batch_word_idxs
    )
    out = _pool_kernel(idx_flat, sq_all, dot_all)
    return out.reshape(BATCH, 1)
